# baseline (speedup 1.0000x reference)
import functools

import jax
import jax.numpy as jnp
from jax import lax
from jax.experimental import pallas as pl
from jax.experimental.pallas import tpu as pltpu
from jax.experimental.pallas import tpu_sc as plsc

N_VOCAB = 1000000
N_H = 64
BATCH = 4096
SEQ = 50

_info = plsc.get_sparse_core_info()
NC, NS = _info.num_cores, _info.num_subcores
NW = NC * NS
RPW = BATCH // NW
NBUF = 4
NPASS = RPW // NBUF

_mesh = plsc.VectorSubcoreMesh(core_axis_name="c", subcore_axis_name="s")


@functools.partial(
    pl.kernel,
    mesh=_mesh,
    out_type=jax.ShapeDtypeStruct((BATCH, SEQ, N_H), jnp.float32),
    scratch_types=[
        pltpu.VMEM((RPW, SEQ), jnp.int32),
        pltpu.VMEM((NBUF, SEQ, N_H), jnp.float32),
        pltpu.SemaphoreType.DMA((NBUF,)),
        pltpu.SemaphoreType.DMA((NBUF,)),
    ],
    compiler_params=pltpu.CompilerParams(use_tc_tiling_on_sc=False),
)
def _gather_kernel(table_hbm, idx_hbm, out_hbm, idx_v, rows_v, gsem, ssem):
    wid = lax.axis_index("s") * NC + lax.axis_index("c")
    base = wid * RPW

    pltpu.sync_copy(idx_hbm.at[pl.ds(base, RPW)], idx_v)

    def fire_gather(s, k):
        pltpu.async_copy(table_hbm.at[idx_v.at[s]], rows_v.at[k], gsem.at[k])

    def wait_gather(s, k):
        pltpu.make_async_copy(table_hbm.at[idx_v.at[s]], rows_v.at[k],
                              gsem.at[k]).wait()

    def fire_write(s, k):
        pltpu.async_copy(rows_v.at[k], out_hbm.at[base + s], ssem.at[k])

    def wait_write(k):
        pltpu.make_async_copy(rows_v.at[k], out_hbm.at[base], ssem.at[k]).wait()

    for k in range(NBUF - 1):
        fire_gather(k, k)

    for k in range(NBUF):
        s = k
        if s + NBUF - 1 < RPW:
            if s >= 1:
                wait_write((k + NBUF - 1) % NBUF)
            fire_gather(s + NBUF - 1, (k + NBUF - 1) % NBUF)
        wait_gather(s, k)
        fire_write(s, k)

    def pass_body(p, _):
        for k in range(NBUF):
            s = p * NBUF + k
            n = (k + NBUF - 1) % NBUF
            wait_write(n)
            fire_gather(s + NBUF - 1, n)
            wait_gather(s, k)
            fire_write(s, k)
        return ()

    lax.fori_loop(1, NPASS - 1, pass_body, ())

    for k in range(NBUF):
        s = (NPASS - 1) * NBUF + k
        if s + NBUF - 1 < RPW:
            wait_write((k + NBUF - 1) % NBUF)
            fire_gather(s + NBUF - 1, (k + NBUF - 1) % NBUF)
        wait_gather(s, k)
        fire_write(s, k)

    for k in range(NBUF):
        wait_write(k)


def kernel(inputs, V, b):
    del b
    return _gather_kernel(V, inputs.astype(jnp.int32))
